# baseline (speedup 1.0000x reference)
import jax
import jax.numpy as jnp
from jax.experimental import pallas as pl

_S, _E, _NQ = 512, 256, 128
_G = 64


def _gnn_body(edges_ref, qe_ref, w0_ref, b0_ref, w1_ref, b1_ref,
              w2_ref, b2_ref, out_ref):
    f32 = jnp.float32
    G = edges_ref.shape[0]
    h0 = jnp.dot(qe_ref[...], w0_ref[...], preferred_element_type=f32)

    row = jax.lax.broadcasted_iota(jnp.int32, (_NQ, _E), 0)
    rr = jax.lax.broadcasted_iota(jnp.int32, (_NQ, _NQ), 0)
    cc = jax.lax.broadcasted_iota(jnp.int32, (_NQ, _NQ), 1)
    eye = (rr == cc).astype(f32)

    dims_ee = (((1,), (1,)), ((), ()))
    As = []
    x1s = []
    for g in range(G):
        src = edges_ref[g, 0:1, :]
        dst = edges_ref[g, 1:2, :]
        oh_src = (src == row).astype(f32)
        oh_dst = (dst == row).astype(f32)
        deg = jnp.sum(oh_dst, axis=1, keepdims=True) + 1.0
        dis = jax.lax.rsqrt(deg)
        a = jax.lax.dot_general(oh_dst * dis, oh_src * dis, dims_ee,
                                preferred_element_type=f32)
        a = a + eye * (dis * dis)
        As.append(a)
        x1s.append(jnp.maximum(jnp.dot(a, h0, preferred_element_type=f32)
                               + b0_ref[...], 0.0))
    x = jnp.concatenate(x1s, axis=0)

    h = jnp.dot(x, w1_ref[...], preferred_element_type=f32)
    x = jnp.concatenate(
        [jnp.maximum(jnp.dot(As[g], h[g * _NQ:(g + 1) * _NQ],
                             preferred_element_type=f32) + b1_ref[...], 0.0)
         for g in range(G)], axis=0)

    h = jnp.dot(x, w2_ref[...], preferred_element_type=f32)
    for g in range(G):
        out_ref[g * _NQ:(g + 1) * _NQ, :] = jnp.maximum(
            jnp.dot(As[g], h[g * _NQ:(g + 1) * _NQ],
                    preferred_element_type=f32) + b2_ref[...], 0.0)


def kernel(slice_matrices, qubit_embeddings, W0, b0, W1, b1, W2, b2):
    edges = slice_matrices.astype(jnp.int32)
    d0, d1 = W0.shape
    d2 = W1.shape[1]
    d3 = W2.shape[1]
    return pl.pallas_call(
        _gnn_body,
        grid=(_S // _G,),
        in_specs=[
            pl.BlockSpec((_G, 2, _E), lambda i: (i, 0, 0)),
            pl.BlockSpec((_NQ, d0), lambda i: (0, 0)),
            pl.BlockSpec((d0, d1), lambda i: (0, 0)),
            pl.BlockSpec((1, d1), lambda i: (0, 0)),
            pl.BlockSpec((d1, d2), lambda i: (0, 0)),
            pl.BlockSpec((1, d2), lambda i: (0, 0)),
            pl.BlockSpec((d2, d3), lambda i: (0, 0)),
            pl.BlockSpec((1, d3), lambda i: (0, 0)),
        ],
        out_specs=pl.BlockSpec((_G * _NQ, d3), lambda i: (i, 0)),
        out_shape=jax.ShapeDtypeStruct((_S * _NQ, d3), jnp.float32),
    )(edges, qubit_embeddings, W0, b0.reshape(1, -1), W1, b1.reshape(1, -1),
      W2, b2.reshape(1, -1))


# speedup vs baseline: 1.4108x; 1.4108x over previous
import jax
import jax.numpy as jnp
from jax.experimental import pallas as pl

_S, _E, _NQ = 512, 256, 128
_G = 64


def _gnn_body(edges_ref, qe_ref, w0_ref, b0_ref, w1_ref, b1_ref,
              w2_ref, b2_ref, out_ref):
    f32 = jnp.float32
    G = edges_ref.shape[0]
    h0 = jnp.dot(qe_ref[...], w0_ref[...], preferred_element_type=f32)

    row = jax.lax.broadcasted_iota(jnp.int32, (_NQ, _E), 0)
    rr = jax.lax.broadcasted_iota(jnp.int32, (_NQ, _NQ), 0)
    cc = jax.lax.broadcasted_iota(jnp.int32, (_NQ, _NQ), 1)
    eye = (rr == cc).astype(f32)

    dims_ee = (((1,), (1,)), ((), ()))
    As = []
    for g in range(G):
        src = edges_ref[g, 0:1, :]
        dst = edges_ref[g, 1:2, :]
        oh_src = (src == row).astype(f32)
        oh_dst = (dst == row).astype(f32)
        deg = jnp.sum(oh_dst, axis=1, keepdims=True) + 1.0
        dis = jax.lax.rsqrt(deg)
        a = jax.lax.dot_general(oh_dst * dis, oh_src * dis, dims_ee,
                                preferred_element_type=f32)
        As.append(a + eye * (dis * dis))

    a_big = jnp.concatenate(As, axis=0)
    x = jnp.maximum(jnp.dot(a_big, h0, preferred_element_type=f32)
                    + b0_ref[...], 0.0)

    h = jnp.dot(x, w1_ref[...], preferred_element_type=f32)
    x = jnp.concatenate(
        [jnp.maximum(jnp.dot(a_big[g * _NQ:(g + 1) * _NQ],
                             h[g * _NQ:(g + 1) * _NQ],
                             preferred_element_type=f32) + b1_ref[...], 0.0)
         for g in range(G)], axis=0)

    h = jnp.dot(x, w2_ref[...], preferred_element_type=f32)
    for g in range(G):
        out_ref[g * _NQ:(g + 1) * _NQ, :] = jnp.maximum(
            jnp.dot(a_big[g * _NQ:(g + 1) * _NQ], h[g * _NQ:(g + 1) * _NQ],
                    preferred_element_type=f32) + b2_ref[...], 0.0)


def kernel(slice_matrices, qubit_embeddings, W0, b0, W1, b1, W2, b2):
    edges = slice_matrices.astype(jnp.int32)
    d0, d1 = W0.shape
    d2 = W1.shape[1]
    d3 = W2.shape[1]
    return pl.pallas_call(
        _gnn_body,
        grid=(_S // _G,),
        in_specs=[
            pl.BlockSpec((_G, 2, _E), lambda i: (i, 0, 0)),
            pl.BlockSpec((_NQ, d0), lambda i: (0, 0)),
            pl.BlockSpec((d0, d1), lambda i: (0, 0)),
            pl.BlockSpec((1, d1), lambda i: (0, 0)),
            pl.BlockSpec((d1, d2), lambda i: (0, 0)),
            pl.BlockSpec((1, d2), lambda i: (0, 0)),
            pl.BlockSpec((d2, d3), lambda i: (0, 0)),
            pl.BlockSpec((1, d3), lambda i: (0, 0)),
        ],
        out_specs=pl.BlockSpec((_G * _NQ, d3), lambda i: (i, 0)),
        out_shape=jax.ShapeDtypeStruct((_S * _NQ, d3), jnp.float32),
    )(edges, qubit_embeddings, W0, b0.reshape(1, -1), W1, b1.reshape(1, -1),
      W2, b2.reshape(1, -1))
